# Initial kernel scaffold; baseline (speedup 1.0000x reference)
import jax
import jax.numpy as jnp
from jax.experimental import pallas as pl


def kernel(x, edge_index, batch, params):
    raise NotImplementedError("write your pallas kernel here")


# speedup vs baseline: 4.0831x; 4.0831x over previous
import functools

import jax
import jax.numpy as jnp
from jax import lax
from jax.experimental import pallas as pl
from jax.experimental.pallas import tpu as pltpu
from jax.experimental.pallas import tpu_sc as plsc

N = 10000
E = 320000
H = 128
G = 128

_NC = 2
_NS = 16
_NW = _NC * _NS
_EPW = E // _NW
_CH = 80
_NCHUNK = _EPW // _CH
_RPS = 624
_RTAIL = N - _NS * _RPS


def _make_seg_sum():
    mesh = plsc.VectorSubcoreMesh(core_axis_name="c", subcore_axis_name="s")

    @functools.partial(
        pl.kernel,
        mesh=mesh,
        out_type=jax.ShapeDtypeStruct((_NC, N, H), jnp.float32),
        scratch_types=[
            pltpu.VMEM((_CH,), jnp.int32),
            pltpu.VMEM((_CH,), jnp.int32),
            pltpu.VMEM((_CH, H), jnp.float32),
            pltpu.VMEM_SHARED((N, H), jnp.float32),
            pltpu.SemaphoreType.DMA,
        ],
    )
    def seg(h_hbm, src_hbm, dst_hbm, out_hbm, sidx, didx, rows, acc, sem):
        cid = lax.axis_index("c")
        sid = lax.axis_index("s")
        wid = cid * _NS + sid
        r0 = sid * _RPS
        pltpu.sync_copy(h_hbm.at[pl.ds(r0, _RPS)], acc.at[pl.ds(r0, _RPS)])

        @pl.when(sid == 0)
        def _():
            pltpu.sync_copy(h_hbm.at[pl.ds(_NS * _RPS, _RTAIL)],
                            acc.at[pl.ds(_NS * _RPS, _RTAIL)])

        plsc.subcore_barrier()

        base = wid * _EPW

        def body(c, carry):
            off = base + c * _CH
            pltpu.sync_copy(src_hbm.at[pl.ds(off, _CH)], sidx)
            pltpu.sync_copy(dst_hbm.at[pl.ds(off, _CH)], didx)
            pltpu.async_copy(h_hbm.at[sidx], rows, sem).wait()
            pltpu.sync_copy(rows, acc.at[didx], add=True)
            return carry

        lax.fori_loop(0, _NCHUNK, body, 0)
        plsc.subcore_barrier()
        pltpu.sync_copy(acc.at[pl.ds(r0, _RPS)], out_hbm.at[cid, pl.ds(r0, _RPS)])

        @pl.when(sid == 0)
        def _():
            pltpu.sync_copy(acc.at[pl.ds(_NS * _RPS, _RTAIL)],
                            out_hbm.at[cid, pl.ds(_NS * _RPS, _RTAIL)])

    return seg


_seg_sum = _make_seg_sum()


_R = 1000


def _proj_body(x_ref, w_ref, b_ref, o_ref):
    acc = jnp.dot(x_ref[...], w_ref[...], preferred_element_type=jnp.float32)
    o_ref[...] = jnp.maximum(acc + b_ref[...], 0.0)


def _proj(x, w, b):
    return pl.pallas_call(
        _proj_body,
        grid=(N // _R,),
        in_specs=[
            pl.BlockSpec((_R, H), lambda i: (i, 0)),
            pl.BlockSpec((H, H), lambda i: (0, 0)),
            pl.BlockSpec((1, H), lambda i: (0, 0)),
        ],
        out_specs=pl.BlockSpec((_R, H), lambda i: (i, 0)),
        out_shape=jax.ShapeDtypeStruct((N, H), jnp.float32),
    )(x, w, b)


def _mlp_body(eps_ref, h_ref, a_ref, w1_ref, b1_ref, w2_ref, b2_ref,
              w3_ref, b3_ref, o_ref, *, residual):
    h = h_ref[...]
    eps = eps_ref[0]
    m = (eps - 1.0) * h + a_ref[0] + a_ref[1]
    m = jnp.dot(m, w1_ref[...], preferred_element_type=jnp.float32) + b1_ref[...]
    m = jnp.maximum(m, 0.0)
    m = jnp.dot(m, w2_ref[...], preferred_element_type=jnp.float32) + b2_ref[...]
    m = jnp.maximum(m, 0.0)
    m = jnp.dot(m, w3_ref[...], preferred_element_type=jnp.float32) + b3_ref[...]
    hn = jnp.maximum(m, 0.0)
    o_ref[...] = h + hn if residual else hn


def _mlp(h, agg2, eps, w1, b1, w2, b2, w3, b3, residual):
    body = functools.partial(_mlp_body, residual=residual)
    return pl.pallas_call(
        body,
        grid=(N // _R,),
        in_specs=[
            pl.BlockSpec(memory_space=pltpu.SMEM),
            pl.BlockSpec((_R, H), lambda i: (i, 0)),
            pl.BlockSpec((_NC, _R, H), lambda i: (0, i, 0)),
            pl.BlockSpec((H, 3 * H), lambda i: (0, 0)),
            pl.BlockSpec((1, 3 * H), lambda i: (0, 0)),
            pl.BlockSpec((3 * H, 2 * H), lambda i: (0, 0)),
            pl.BlockSpec((1, 2 * H), lambda i: (0, 0)),
            pl.BlockSpec((2 * H, H), lambda i: (0, 0)),
            pl.BlockSpec((1, H), lambda i: (0, 0)),
        ],
        out_specs=pl.BlockSpec((_R, H), lambda i: (i, 0)),
        out_shape=jax.ShapeDtypeStruct((N, H), jnp.float32),
    )(eps, h, agg2, w1, b1, w2, b2, w3, b3)


def _pool_body(h_ref, b_ref, w1_ref, b1_ref, w2_ref, b2_ref, w3_ref, b3_ref,
               o_ref, pmax_ref):
    h = h_ref[...]
    bid = b_ref[...]
    gids = lax.broadcasted_iota(jnp.int32, (N, G), 1)
    oh = (bid == gids).astype(jnp.float32)
    psum = lax.dot_general(oh, h, (((0,), (0,)), ((), ())),
                           preferred_element_type=jnp.float32)
    cnt = jnp.sum(oh, axis=0)[:, None]
    pmean = psum / jnp.maximum(cnt, 1.0)

    def body(g, carry):
        mask = bid == g
        hm = jnp.where(mask, h, -jnp.inf)
        row = jnp.max(hm, axis=0)
        pmax_ref[pl.ds(g, 1), :] = row[None, :]
        return carry

    lax.fori_loop(0, G, body, 0)
    pmax = pmax_ref[...]
    pmax = jnp.where(jnp.isfinite(pmax), pmax, 0.0)
    ge = jnp.concatenate([psum, pmean, pmax], axis=1)
    f = jnp.dot(ge, w1_ref[...], preferred_element_type=jnp.float32) + b1_ref[...]
    f = jnp.maximum(f, 0.0)
    f = jnp.dot(f, w2_ref[...], preferred_element_type=jnp.float32) + b2_ref[...]
    f = jnp.maximum(f, 0.0)
    o_ref[...] = jnp.dot(f, w3_ref[...], preferred_element_type=jnp.float32) + b3_ref[...]


def _pool_cls(h, batch2d, w1, b1, w2, b2, w3, b3):
    return pl.pallas_call(
        _pool_body,
        out_shape=jax.ShapeDtypeStruct((G, 6), jnp.float32),
        scratch_shapes=[pltpu.VMEM((G, H), jnp.float32)],
    )(h, batch2d, w1, b1, w2, b2, w3, b3)


def kernel(x, edge_index, batch, params):
    inv_s = 1.0 / jnp.sqrt(jnp.float32(1.0) + 1e-5)

    def fold(w, b, g, be):
        scale = g * inv_s
        return w * scale[None, :], (b * scale + be)[None, :]

    src = edge_index[0]
    dst = edge_index[1]

    w_in, b_in = fold(params['in_W'], params['in_b'], params['in_g'], params['in_be'])
    h = _proj(x, w_in, b_in)

    for i in range(5):
        w1, b1 = fold(params['c%d_W1' % i], params['c%d_b1' % i],
                      params['c%d_g1' % i], params['c%d_be1' % i])
        w2, b2 = fold(params['c%d_W2' % i], params['c%d_b2' % i],
                      params['c%d_g2' % i], params['c%d_be2' % i])
        w3, b3 = fold(params['c%d_W3' % i], params['c%d_b3' % i],
                      params['n%d_g' % i], params['n%d_be' % i])
        eps = params['c%d_eps' % i].reshape(1)
        agg2 = _seg_sum(h, src, dst)
        h = _mlp(h, agg2, eps, w1, b1, w2, b2, w3, b3, residual=(i > 0))

    wc1, bc1 = fold(params['cl_W1'], params['cl_b1'], params['cl_g1'], params['cl_be1'])
    wc2, bc2 = fold(params['cl_W2'], params['cl_b2'], params['cl_g2'], params['cl_be2'])
    fw = params['f_W']
    fb = params['f_b'][None, :]
    return _pool_cls(h, batch.reshape(N, 1), wc1, bc1, wc2, bc2, fw, fb)


# speedup vs baseline: 6.6019x; 1.6169x over previous
import functools

import jax
import jax.numpy as jnp
from jax import lax
from jax.experimental import pallas as pl
from jax.experimental.pallas import tpu as pltpu
from jax.experimental.pallas import tpu_sc as plsc

N = 10000
E = 320000
H = 128
G = 128

_NC = 2
_NS = 16
_NW = _NC * _NS
_EPW = E // _NW
_CH = 128
_NPAIR = 39
_ETAIL = _EPW - 2 * _NPAIR * _CH
_RPS = 624
_RTAIL = N - _NS * _RPS


def _make_seg_sum():
    mesh = plsc.VectorSubcoreMesh(core_axis_name="c", subcore_axis_name="s")

    @functools.partial(
        pl.kernel,
        mesh=mesh,
        out_type=jax.ShapeDtypeStruct((_NC, N, H), jnp.float32),
        scratch_types=[
            pltpu.VMEM((_CH,), jnp.int32),
            pltpu.VMEM((_CH,), jnp.int32),
            pltpu.VMEM((_CH,), jnp.int32),
            pltpu.VMEM((_CH,), jnp.int32),
            pltpu.VMEM((_CH, H), jnp.float32),
            pltpu.VMEM((_CH, H), jnp.float32),
            pltpu.VMEM((_ETAIL,), jnp.int32),
            pltpu.VMEM((_ETAIL,), jnp.int32),
            pltpu.VMEM((_ETAIL, H), jnp.float32),
            pltpu.VMEM_SHARED((N, H), jnp.float32),
            pltpu.SemaphoreType.DMA,
            pltpu.SemaphoreType.DMA,
            pltpu.SemaphoreType.DMA,
            pltpu.SemaphoreType.DMA,
        ],
    )
    def seg(h_hbm, src_hbm, dst_hbm, out_hbm,
            sidx0, didx0, sidx1, didx1, rows0, rows1,
            sidxt, didxt, rowst, acc, semi0, semi1, semg0, semg1):
        cid = lax.axis_index("c")
        sid = lax.axis_index("s")
        wid = cid * _NS + sid
        r0 = sid * _RPS
        pltpu.sync_copy(h_hbm.at[pl.ds(r0, _RPS)], acc.at[pl.ds(r0, _RPS)])

        @pl.when(sid == 0)
        def _():
            pltpu.sync_copy(h_hbm.at[pl.ds(_NS * _RPS, _RTAIL)],
                            acc.at[pl.ds(_NS * _RPS, _RTAIL)])

        plsc.subcore_barrier()

        base = wid * _EPW

        def body(j, carry):
            off0 = base + (2 * j) * _CH
            off1 = off0 + _CH
            i0s = pltpu.async_copy(src_hbm.at[pl.ds(off0, _CH)], sidx0, semi0)
            i0d = pltpu.async_copy(dst_hbm.at[pl.ds(off0, _CH)], didx0, semi0)
            i1s = pltpu.async_copy(src_hbm.at[pl.ds(off1, _CH)], sidx1, semi1)
            i1d = pltpu.async_copy(dst_hbm.at[pl.ds(off1, _CH)], didx1, semi1)
            i0s.wait()
            i0d.wait()
            g0 = pltpu.async_copy(h_hbm.at[sidx0], rows0, semg0)
            i1s.wait()
            i1d.wait()
            g1 = pltpu.async_copy(h_hbm.at[sidx1], rows1, semg1)
            g0.wait()
            pltpu.sync_copy(rows0, acc.at[didx0], add=True)
            g1.wait()
            pltpu.sync_copy(rows1, acc.at[didx1], add=True)
            return carry

        lax.fori_loop(0, _NPAIR, body, 0)

        offt = base + 2 * _NPAIR * _CH
        pltpu.sync_copy(src_hbm.at[pl.ds(offt, _ETAIL)], sidxt)
        pltpu.sync_copy(dst_hbm.at[pl.ds(offt, _ETAIL)], didxt)
        pltpu.async_copy(h_hbm.at[sidxt], rowst, semg0).wait()
        pltpu.sync_copy(rowst, acc.at[didxt], add=True)
        plsc.subcore_barrier()
        pltpu.sync_copy(acc.at[pl.ds(r0, _RPS)], out_hbm.at[cid, pl.ds(r0, _RPS)])

        @pl.when(sid == 0)
        def _():
            pltpu.sync_copy(acc.at[pl.ds(_NS * _RPS, _RTAIL)],
                            out_hbm.at[cid, pl.ds(_NS * _RPS, _RTAIL)])

    return seg


_seg_sum = _make_seg_sum()


_R = 1000


def _proj_body(x_ref, w_ref, b_ref, o_ref):
    acc = jnp.dot(x_ref[...], w_ref[...], preferred_element_type=jnp.float32)
    o_ref[...] = jnp.maximum(acc + b_ref[...], 0.0)


def _proj(x, w, b):
    return pl.pallas_call(
        _proj_body,
        grid=(N // _R,),
        in_specs=[
            pl.BlockSpec((_R, H), lambda i: (i, 0)),
            pl.BlockSpec((H, H), lambda i: (0, 0)),
            pl.BlockSpec((1, H), lambda i: (0, 0)),
        ],
        out_specs=pl.BlockSpec((_R, H), lambda i: (i, 0)),
        out_shape=jax.ShapeDtypeStruct((N, H), jnp.float32),
    )(x, w, b)


def _mlp_body(eps_ref, h_ref, a_ref, w1_ref, b1_ref, w2_ref, b2_ref,
              w3_ref, b3_ref, o_ref, *, residual):
    h = h_ref[...]
    eps = eps_ref[0]
    m = (eps - 1.0) * h + a_ref[0] + a_ref[1]
    m = jnp.dot(m, w1_ref[...], preferred_element_type=jnp.float32) + b1_ref[...]
    m = jnp.maximum(m, 0.0)
    m = jnp.dot(m, w2_ref[...], preferred_element_type=jnp.float32) + b2_ref[...]
    m = jnp.maximum(m, 0.0)
    m = jnp.dot(m, w3_ref[...], preferred_element_type=jnp.float32) + b3_ref[...]
    hn = jnp.maximum(m, 0.0)
    o_ref[...] = h + hn if residual else hn


def _mlp(h, agg2, eps, w1, b1, w2, b2, w3, b3, residual):
    body = functools.partial(_mlp_body, residual=residual)
    return pl.pallas_call(
        body,
        grid=(N // _R,),
        in_specs=[
            pl.BlockSpec(memory_space=pltpu.SMEM),
            pl.BlockSpec((_R, H), lambda i: (i, 0)),
            pl.BlockSpec((_NC, _R, H), lambda i: (0, i, 0)),
            pl.BlockSpec((H, 3 * H), lambda i: (0, 0)),
            pl.BlockSpec((1, 3 * H), lambda i: (0, 0)),
            pl.BlockSpec((3 * H, 2 * H), lambda i: (0, 0)),
            pl.BlockSpec((1, 2 * H), lambda i: (0, 0)),
            pl.BlockSpec((2 * H, H), lambda i: (0, 0)),
            pl.BlockSpec((1, H), lambda i: (0, 0)),
        ],
        out_specs=pl.BlockSpec((_R, H), lambda i: (i, 0)),
        out_shape=jax.ShapeDtypeStruct((N, H), jnp.float32),
    )(eps, h, agg2, w1, b1, w2, b2, w3, b3)


def _pool_body(h_ref, b_ref, w1_ref, b1_ref, w2_ref, b2_ref, w3_ref, b3_ref,
               o_ref, pmax_ref):
    h = h_ref[...]
    bid = b_ref[...]
    gids = lax.broadcasted_iota(jnp.int32, (N, G), 1)
    oh = (bid == gids).astype(jnp.float32)
    psum = lax.dot_general(oh, h, (((0,), (0,)), ((), ())),
                           preferred_element_type=jnp.float32)
    cnt = jnp.sum(oh, axis=0)[:, None]
    pmean = psum / jnp.maximum(cnt, 1.0)

    def body(g, carry):
        mask = bid == g
        hm = jnp.where(mask, h, -jnp.inf)
        row = jnp.max(hm, axis=0)
        pmax_ref[pl.ds(g, 1), :] = row[None, :]
        return carry

    lax.fori_loop(0, G, body, 0)
    pmax = pmax_ref[...]
    pmax = jnp.where(jnp.isfinite(pmax), pmax, 0.0)
    ge = jnp.concatenate([psum, pmean, pmax], axis=1)
    f = jnp.dot(ge, w1_ref[...], preferred_element_type=jnp.float32) + b1_ref[...]
    f = jnp.maximum(f, 0.0)
    f = jnp.dot(f, w2_ref[...], preferred_element_type=jnp.float32) + b2_ref[...]
    f = jnp.maximum(f, 0.0)
    o_ref[...] = jnp.dot(f, w3_ref[...], preferred_element_type=jnp.float32) + b3_ref[...]


def _pool_cls(h, batch2d, w1, b1, w2, b2, w3, b3):
    return pl.pallas_call(
        _pool_body,
        out_shape=jax.ShapeDtypeStruct((G, 6), jnp.float32),
        scratch_shapes=[pltpu.VMEM((G, H), jnp.float32)],
    )(h, batch2d, w1, b1, w2, b2, w3, b3)


def kernel(x, edge_index, batch, params):
    inv_s = 1.0 / jnp.sqrt(jnp.float32(1.0) + 1e-5)

    def fold(w, b, g, be):
        scale = g * inv_s
        return w * scale[None, :], (b * scale + be)[None, :]

    src = edge_index[0]
    dst = edge_index[1]

    w_in, b_in = fold(params['in_W'], params['in_b'], params['in_g'], params['in_be'])
    h = _proj(x, w_in, b_in)

    for i in range(5):
        w1, b1 = fold(params['c%d_W1' % i], params['c%d_b1' % i],
                      params['c%d_g1' % i], params['c%d_be1' % i])
        w2, b2 = fold(params['c%d_W2' % i], params['c%d_b2' % i],
                      params['c%d_g2' % i], params['c%d_be2' % i])
        w3, b3 = fold(params['c%d_W3' % i], params['c%d_b3' % i],
                      params['n%d_g' % i], params['n%d_be' % i])
        eps = params['c%d_eps' % i].reshape(1)
        agg2 = _seg_sum(h, src, dst)
        h = _mlp(h, agg2, eps, w1, b1, w2, b2, w3, b3, residual=(i > 0))

    wc1, bc1 = fold(params['cl_W1'], params['cl_b1'], params['cl_g1'], params['cl_be1'])
    wc2, bc2 = fold(params['cl_W2'], params['cl_b2'], params['cl_g2'], params['cl_be2'])
    fw = params['f_W']
    fb = params['f_b'][None, :]
    return _pool_cls(h, batch.reshape(N, 1), wc1, bc1, wc2, bc2, fw, fb)


# speedup vs baseline: 8.3706x; 1.2679x over previous
import functools

import jax
import jax.numpy as jnp
from jax import lax
from jax.experimental import pallas as pl
from jax.experimental.pallas import tpu as pltpu
from jax.experimental.pallas import tpu_sc as plsc

N = 10000
E = 320000
H = 128
G = 128

_NC = 2
_NS = 16
_NW = _NC * _NS
_CH = 80
_NCK = 125
_RPS = 624
_RTAIL = N - _NS * _RPS


def _make_seg_sum():
    mesh = plsc.VectorSubcoreMesh(core_axis_name="c", subcore_axis_name="s")

    @functools.partial(
        pl.kernel,
        mesh=mesh,
        out_type=jax.ShapeDtypeStruct((_NC, N, H), jnp.float32),
        scratch_types=[
            pltpu.VMEM((_NCK * _CH,), jnp.int32),
            pltpu.VMEM((_NCK, _CH), jnp.int32),
            pltpu.VMEM((_CH, H), jnp.float32),
            pltpu.VMEM((_CH, H), jnp.float32),
            pltpu.VMEM_SHARED((N, H), jnp.float32),
            pltpu.SemaphoreType.DMA,
            pltpu.SemaphoreType.DMA,
        ],
    )
    def seg(h_hbm, src3_hbm, dst3_hbm, out_hbm,
            sidx, didx, rows0, rows1, acc, semg0, semg1):
        cid = lax.axis_index("c")
        sid = lax.axis_index("s")
        wid = cid * _NS + sid
        r0 = sid * _RPS
        pltpu.sync_copy(src3_hbm.at[pl.ds(wid * _NCK * _CH, _NCK * _CH)], sidx)
        pltpu.sync_copy(dst3_hbm.at[wid], didx)
        pltpu.sync_copy(h_hbm.at[pl.ds(r0, _RPS)], acc.at[pl.ds(r0, _RPS)])

        @pl.when(sid == 0)
        def _():
            pltpu.sync_copy(h_hbm.at[pl.ds(_NS * _RPS, _RTAIL)],
                            acc.at[pl.ds(_NS * _RPS, _RTAIL)])

        plsc.subcore_barrier()

        pltpu.async_copy(h_hbm.at[sidx.at[pl.ds(0, _CH)]], rows0, semg0)
        pltpu.async_copy(h_hbm.at[sidx.at[pl.ds(_CH, _CH)]], rows1, semg1)

        def body(j, carry):
            c0 = 2 * j
            pltpu.make_async_copy(h_hbm.at[sidx.at[pl.ds(c0 * _CH, _CH)]], rows0, semg0).wait()
            pltpu.sync_copy(rows0, acc.at[didx.at[c0]], add=True)
            pltpu.async_copy(h_hbm.at[sidx.at[pl.ds((c0 + 2) * _CH, _CH)]], rows0, semg0)
            pltpu.make_async_copy(h_hbm.at[sidx.at[pl.ds((c0 + 1) * _CH, _CH)]], rows1, semg1).wait()
            pltpu.sync_copy(rows1, acc.at[didx.at[c0 + 1]], add=True)
            pltpu.async_copy(h_hbm.at[sidx.at[pl.ds((c0 + 3) * _CH, _CH)]], rows1, semg1)
            return carry

        lax.fori_loop(0, (_NCK - 3) // 2, body, 0)

        pltpu.make_async_copy(h_hbm.at[sidx.at[pl.ds((_NCK - 3) * _CH, _CH)]], rows0, semg0).wait()
        pltpu.sync_copy(rows0, acc.at[didx.at[_NCK - 3]], add=True)
        pltpu.async_copy(h_hbm.at[sidx.at[pl.ds((_NCK - 1) * _CH, _CH)]], rows0, semg0)
        pltpu.make_async_copy(h_hbm.at[sidx.at[pl.ds((_NCK - 2) * _CH, _CH)]], rows1, semg1).wait()
        pltpu.sync_copy(rows1, acc.at[didx.at[_NCK - 2]], add=True)
        pltpu.make_async_copy(h_hbm.at[sidx.at[pl.ds((_NCK - 1) * _CH, _CH)]], rows0, semg0).wait()
        pltpu.sync_copy(rows0, acc.at[didx.at[_NCK - 1]], add=True)

        plsc.subcore_barrier()
        pltpu.sync_copy(acc.at[pl.ds(r0, _RPS)], out_hbm.at[cid, pl.ds(r0, _RPS)])

        @pl.when(sid == 0)
        def _():
            pltpu.sync_copy(acc.at[pl.ds(_NS * _RPS, _RTAIL)],
                            out_hbm.at[cid, pl.ds(_NS * _RPS, _RTAIL)])

    return seg


_seg_sum = _make_seg_sum()


_R = 1000


def _proj_body(x_ref, w_ref, b_ref, o_ref):
    acc = jnp.dot(x_ref[...], w_ref[...], preferred_element_type=jnp.float32)
    o_ref[...] = jnp.maximum(acc + b_ref[...], 0.0)


def _proj(x, w, b):
    return pl.pallas_call(
        _proj_body,
        grid=(N // _R,),
        in_specs=[
            pl.BlockSpec((_R, H), lambda i: (i, 0)),
            pl.BlockSpec((H, H), lambda i: (0, 0)),
            pl.BlockSpec((1, H), lambda i: (0, 0)),
        ],
        out_specs=pl.BlockSpec((_R, H), lambda i: (i, 0)),
        out_shape=jax.ShapeDtypeStruct((N, H), jnp.float32),
    )(x, w, b)


def _mlp_body(eps_ref, h_ref, a_ref, w1_ref, b1_ref, w2_ref, b2_ref,
              w3_ref, b3_ref, o_ref, *, residual):
    h = h_ref[...]
    eps = eps_ref[0]
    m = (eps - 1.0) * h + a_ref[0] + a_ref[1]
    m = jnp.dot(m, w1_ref[...], preferred_element_type=jnp.float32) + b1_ref[...]
    m = jnp.maximum(m, 0.0)
    m = jnp.dot(m, w2_ref[...], preferred_element_type=jnp.float32) + b2_ref[...]
    m = jnp.maximum(m, 0.0)
    m = jnp.dot(m, w3_ref[...], preferred_element_type=jnp.float32) + b3_ref[...]
    hn = jnp.maximum(m, 0.0)
    o_ref[...] = h + hn if residual else hn


def _mlp(h, agg2, eps, w1, b1, w2, b2, w3, b3, residual):
    body = functools.partial(_mlp_body, residual=residual)
    return pl.pallas_call(
        body,
        grid=(N // _R,),
        in_specs=[
            pl.BlockSpec(memory_space=pltpu.SMEM),
            pl.BlockSpec((_R, H), lambda i: (i, 0)),
            pl.BlockSpec((_NC, _R, H), lambda i: (0, i, 0)),
            pl.BlockSpec((H, 3 * H), lambda i: (0, 0)),
            pl.BlockSpec((1, 3 * H), lambda i: (0, 0)),
            pl.BlockSpec((3 * H, 2 * H), lambda i: (0, 0)),
            pl.BlockSpec((1, 2 * H), lambda i: (0, 0)),
            pl.BlockSpec((2 * H, H), lambda i: (0, 0)),
            pl.BlockSpec((1, H), lambda i: (0, 0)),
        ],
        out_specs=pl.BlockSpec((_R, H), lambda i: (i, 0)),
        out_shape=jax.ShapeDtypeStruct((N, H), jnp.float32),
    )(eps, h, agg2, w1, b1, w2, b2, w3, b3)


def _pool_body(h_ref, b_ref, w1_ref, b1_ref, w2_ref, b2_ref, w3_ref, b3_ref,
               o_ref, pmax_ref):
    h = h_ref[...]
    bid = b_ref[...]
    gids = lax.broadcasted_iota(jnp.int32, (N, G), 1)
    oh = (bid == gids).astype(jnp.float32)
    psum = lax.dot_general(oh, h, (((0,), (0,)), ((), ())),
                           preferred_element_type=jnp.float32)
    cnt = jnp.sum(oh, axis=0)[:, None]
    pmean = psum / jnp.maximum(cnt, 1.0)

    def body(g, carry):
        mask = bid == g
        hm = jnp.where(mask, h, -jnp.inf)
        row = jnp.max(hm, axis=0)
        pmax_ref[pl.ds(g, 1), :] = row[None, :]
        return carry

    lax.fori_loop(0, G, body, 0)
    pmax = pmax_ref[...]
    pmax = jnp.where(jnp.isfinite(pmax), pmax, 0.0)
    ge = jnp.concatenate([psum, pmean, pmax], axis=1)
    f = jnp.dot(ge, w1_ref[...], preferred_element_type=jnp.float32) + b1_ref[...]
    f = jnp.maximum(f, 0.0)
    f = jnp.dot(f, w2_ref[...], preferred_element_type=jnp.float32) + b2_ref[...]
    f = jnp.maximum(f, 0.0)
    o_ref[...] = jnp.dot(f, w3_ref[...], preferred_element_type=jnp.float32) + b3_ref[...]


def _pool_cls(h, batch2d, w1, b1, w2, b2, w3, b3):
    return pl.pallas_call(
        _pool_body,
        out_shape=jax.ShapeDtypeStruct((G, 6), jnp.float32),
        scratch_shapes=[pltpu.VMEM((G, H), jnp.float32)],
    )(h, batch2d, w1, b1, w2, b2, w3, b3)


def kernel(x, edge_index, batch, params):
    inv_s = 1.0 / jnp.sqrt(jnp.float32(1.0) + 1e-5)

    def fold(w, b, g, be):
        scale = g * inv_s
        return w * scale[None, :], (b * scale + be)[None, :]

    src3 = edge_index[0]
    dst3 = edge_index[1].reshape(_NW, _NCK, _CH)

    w_in, b_in = fold(params['in_W'], params['in_b'], params['in_g'], params['in_be'])
    h = _proj(x, w_in, b_in)

    for i in range(5):
        w1, b1 = fold(params['c%d_W1' % i], params['c%d_b1' % i],
                      params['c%d_g1' % i], params['c%d_be1' % i])
        w2, b2 = fold(params['c%d_W2' % i], params['c%d_b2' % i],
                      params['c%d_g2' % i], params['c%d_be2' % i])
        w3, b3 = fold(params['c%d_W3' % i], params['c%d_b3' % i],
                      params['n%d_g' % i], params['n%d_be' % i])
        eps = params['c%d_eps' % i].reshape(1)
        agg2 = _seg_sum(h, src3, dst3)
        h = _mlp(h, agg2, eps, w1, b1, w2, b2, w3, b3, residual=(i > 0))

    wc1, bc1 = fold(params['cl_W1'], params['cl_b1'], params['cl_g1'], params['cl_be1'])
    wc2, bc2 = fold(params['cl_W2'], params['cl_b2'], params['cl_g2'], params['cl_be2'])
    fw = params['f_W']
    fb = params['f_b'][None, :]
    return _pool_cls(h, batch.reshape(N, 1), wc1, bc1, wc2, bc2, fw, fb)


# speedup vs baseline: 8.6413x; 1.0323x over previous
import functools

import jax
import jax.numpy as jnp
from jax import lax
from jax.experimental import pallas as pl
from jax.experimental.pallas import tpu as pltpu
from jax.experimental.pallas import tpu_sc as plsc

N = 10000
E = 320000
H = 128
G = 128

_NC = 2
_NS = 16
_NW = _NC * _NS
_CH = 80
_NCK = 125
_RPS = 624
_RTAIL = N - _NS * _RPS


def _make_seg_sum():
    mesh = plsc.VectorSubcoreMesh(core_axis_name="c", subcore_axis_name="s")

    @functools.partial(
        pl.kernel,
        mesh=mesh,
        out_type=jax.ShapeDtypeStruct((_NC, N, H), jnp.float32),
        scratch_types=[
            pltpu.VMEM((_NCK * _CH,), jnp.int32),
            pltpu.VMEM((_NCK, _CH), jnp.int32),
            pltpu.VMEM((_CH, H), jnp.float32),
            pltpu.VMEM((_CH, H), jnp.float32),
            pltpu.VMEM_SHARED((N, H), jnp.float32),
            pltpu.SemaphoreType.DMA,
            pltpu.SemaphoreType.DMA,
        ],
    )
    def seg(h_hbm, src3_hbm, dst3_hbm, out_hbm,
            sidx, didx, rows0, rows1, acc, semg0, semg1):
        cid = lax.axis_index("c")
        sid = lax.axis_index("s")
        wid = cid * _NS + sid
        r0 = sid * _RPS
        c1 = pltpu.async_copy(src3_hbm.at[pl.ds(wid * _NCK * _CH, _NCK * _CH)],
                              sidx, semg0)
        c2 = pltpu.async_copy(dst3_hbm.at[wid], didx, semg1)
        c3 = pltpu.async_copy(h_hbm.at[pl.ds(r0, _RPS)],
                              acc.at[pl.ds(r0, _RPS)], semg0)

        @pl.when(sid == 0)
        def _():
            pltpu.async_copy(h_hbm.at[pl.ds(_NS * _RPS, _RTAIL)],
                             acc.at[pl.ds(_NS * _RPS, _RTAIL)], semg1).wait()

        c1.wait()
        c2.wait()
        c3.wait()
        plsc.subcore_barrier()

        pltpu.async_copy(h_hbm.at[sidx.at[pl.ds(0, _CH)]], rows0, semg0)
        pltpu.async_copy(h_hbm.at[sidx.at[pl.ds(_CH, _CH)]], rows1, semg1)

        def body(j, carry):
            c0 = 2 * j
            pltpu.make_async_copy(h_hbm.at[sidx.at[pl.ds(c0 * _CH, _CH)]], rows0, semg0).wait()
            pltpu.sync_copy(rows0, acc.at[didx.at[c0]], add=True)
            pltpu.async_copy(h_hbm.at[sidx.at[pl.ds((c0 + 2) * _CH, _CH)]], rows0, semg0)
            pltpu.make_async_copy(h_hbm.at[sidx.at[pl.ds((c0 + 1) * _CH, _CH)]], rows1, semg1).wait()
            pltpu.sync_copy(rows1, acc.at[didx.at[c0 + 1]], add=True)
            pltpu.async_copy(h_hbm.at[sidx.at[pl.ds((c0 + 3) * _CH, _CH)]], rows1, semg1)
            return carry

        lax.fori_loop(0, (_NCK - 3) // 2, body, 0)

        pltpu.make_async_copy(h_hbm.at[sidx.at[pl.ds((_NCK - 3) * _CH, _CH)]], rows0, semg0).wait()
        pltpu.sync_copy(rows0, acc.at[didx.at[_NCK - 3]], add=True)
        pltpu.async_copy(h_hbm.at[sidx.at[pl.ds((_NCK - 1) * _CH, _CH)]], rows0, semg0)
        pltpu.make_async_copy(h_hbm.at[sidx.at[pl.ds((_NCK - 2) * _CH, _CH)]], rows1, semg1).wait()
        pltpu.sync_copy(rows1, acc.at[didx.at[_NCK - 2]], add=True)
        pltpu.make_async_copy(h_hbm.at[sidx.at[pl.ds((_NCK - 1) * _CH, _CH)]], rows0, semg0).wait()
        pltpu.sync_copy(rows0, acc.at[didx.at[_NCK - 1]], add=True)

        plsc.subcore_barrier()
        pltpu.sync_copy(acc.at[pl.ds(r0, _RPS)], out_hbm.at[cid, pl.ds(r0, _RPS)])

        @pl.when(sid == 0)
        def _():
            pltpu.sync_copy(acc.at[pl.ds(_NS * _RPS, _RTAIL)],
                            out_hbm.at[cid, pl.ds(_NS * _RPS, _RTAIL)])

    return seg


_seg_sum = _make_seg_sum()


_R = 2000


def _proj_body(x_ref, w_ref, b_ref, o_ref):
    acc = jnp.dot(x_ref[...], w_ref[...], preferred_element_type=jnp.float32)
    o_ref[...] = jnp.maximum(acc + b_ref[...], 0.0)


def _proj(x, w, b):
    return pl.pallas_call(
        _proj_body,
        grid=(N // _R,),
        in_specs=[
            pl.BlockSpec((_R, H), lambda i: (i, 0)),
            pl.BlockSpec((H, H), lambda i: (0, 0)),
            pl.BlockSpec((1, H), lambda i: (0, 0)),
        ],
        out_specs=pl.BlockSpec((_R, H), lambda i: (i, 0)),
        out_shape=jax.ShapeDtypeStruct((N, H), jnp.float32),
    )(x, w, b)


def _mlp_body(eps_ref, h_ref, a_ref, w1_ref, b1_ref, w2_ref, b2_ref,
              w3_ref, b3_ref, o_ref, *, residual):
    h = h_ref[...]
    eps = eps_ref[0]
    m = (eps - 1.0) * h + a_ref[0] + a_ref[1]
    m = jnp.dot(m, w1_ref[...], preferred_element_type=jnp.float32) + b1_ref[...]
    m = jnp.maximum(m, 0.0)
    m = jnp.dot(m, w2_ref[...], preferred_element_type=jnp.float32) + b2_ref[...]
    m = jnp.maximum(m, 0.0)
    m = jnp.dot(m, w3_ref[...], preferred_element_type=jnp.float32) + b3_ref[...]
    hn = jnp.maximum(m, 0.0)
    o_ref[...] = h + hn if residual else hn


def _mlp(h, agg2, eps, w1, b1, w2, b2, w3, b3, residual):
    body = functools.partial(_mlp_body, residual=residual)
    return pl.pallas_call(
        body,
        grid=(N // _R,),
        in_specs=[
            pl.BlockSpec(memory_space=pltpu.SMEM),
            pl.BlockSpec((_R, H), lambda i: (i, 0)),
            pl.BlockSpec((_NC, _R, H), lambda i: (0, i, 0)),
            pl.BlockSpec((H, 3 * H), lambda i: (0, 0)),
            pl.BlockSpec((1, 3 * H), lambda i: (0, 0)),
            pl.BlockSpec((3 * H, 2 * H), lambda i: (0, 0)),
            pl.BlockSpec((1, 2 * H), lambda i: (0, 0)),
            pl.BlockSpec((2 * H, H), lambda i: (0, 0)),
            pl.BlockSpec((1, H), lambda i: (0, 0)),
        ],
        out_specs=pl.BlockSpec((_R, H), lambda i: (i, 0)),
        out_shape=jax.ShapeDtypeStruct((N, H), jnp.float32),
    )(eps, h, agg2, w1, b1, w2, b2, w3, b3)


def _pool_body(h_ref, b_ref, w1_ref, b1_ref, w2_ref, b2_ref, w3_ref, b3_ref,
               o_ref, pmax_ref):
    h = h_ref[...]
    bid = b_ref[...]
    gids = lax.broadcasted_iota(jnp.int32, (N, G), 1)
    oh = (bid == gids).astype(jnp.float32)
    psum = lax.dot_general(oh, h, (((0,), (0,)), ((), ())),
                           preferred_element_type=jnp.float32)
    cnt = jnp.sum(oh, axis=0)[:, None]
    pmean = psum / jnp.maximum(cnt, 1.0)

    def body(g, carry):
        mask = bid == g
        hm = jnp.where(mask, h, -jnp.inf)
        row = jnp.max(hm, axis=0)
        pmax_ref[pl.ds(g, 1), :] = row[None, :]
        return carry

    lax.fori_loop(0, G, body, 0)
    pmax = pmax_ref[...]
    pmax = jnp.where(jnp.isfinite(pmax), pmax, 0.0)
    ge = jnp.concatenate([psum, pmean, pmax], axis=1)
    f = jnp.dot(ge, w1_ref[...], preferred_element_type=jnp.float32) + b1_ref[...]
    f = jnp.maximum(f, 0.0)
    f = jnp.dot(f, w2_ref[...], preferred_element_type=jnp.float32) + b2_ref[...]
    f = jnp.maximum(f, 0.0)
    o_ref[...] = jnp.dot(f, w3_ref[...], preferred_element_type=jnp.float32) + b3_ref[...]


def _pool_cls(h, batch2d, w1, b1, w2, b2, w3, b3):
    return pl.pallas_call(
        _pool_body,
        out_shape=jax.ShapeDtypeStruct((G, 6), jnp.float32),
        scratch_shapes=[pltpu.VMEM((G, H), jnp.float32)],
    )(h, batch2d, w1, b1, w2, b2, w3, b3)


def kernel(x, edge_index, batch, params):
    inv_s = 1.0 / jnp.sqrt(jnp.float32(1.0) + 1e-5)

    def fold(w, b, g, be):
        scale = g * inv_s
        return w * scale[None, :], (b * scale + be)[None, :]

    src3 = edge_index[0]
    dst3 = edge_index[1].reshape(_NW, _NCK, _CH)

    w_in, b_in = fold(params['in_W'], params['in_b'], params['in_g'], params['in_be'])
    h = _proj(x, w_in, b_in)

    for i in range(5):
        w1, b1 = fold(params['c%d_W1' % i], params['c%d_b1' % i],
                      params['c%d_g1' % i], params['c%d_be1' % i])
        w2, b2 = fold(params['c%d_W2' % i], params['c%d_b2' % i],
                      params['c%d_g2' % i], params['c%d_be2' % i])
        w3, b3 = fold(params['c%d_W3' % i], params['c%d_b3' % i],
                      params['n%d_g' % i], params['n%d_be' % i])
        eps = params['c%d_eps' % i].reshape(1)
        agg2 = _seg_sum(h, src3, dst3)
        h = _mlp(h, agg2, eps, w1, b1, w2, b2, w3, b3, residual=(i > 0))

    wc1, bc1 = fold(params['cl_W1'], params['cl_b1'], params['cl_g1'], params['cl_be1'])
    wc2, bc2 = fold(params['cl_W2'], params['cl_b2'], params['cl_g2'], params['cl_be2'])
    fw = params['f_W']
    fb = params['f_b'][None, :]
    return _pool_cls(h, batch.reshape(N, 1), wc1, bc1, wc2, bc2, fw, fb)
